# baseline (speedup 1.0000x reference)
import functools

import jax
import jax.numpy as jnp
from jax import lax
from jax.experimental import pallas as pl
from jax.experimental.pallas import tpu as pltpu
from jax.experimental.pallas import tpu_sc as plsc

B = 1024
NJ = 100
NM = 20
NT = 10
NO = 500
NOP = 512
NA = 1 + NJ * NM * NT
NW = 32
SPW = B // NW
BIG = 1e9


_BPB = 256


def _bp_body(adj_ref, o_ref):
    a = adj_ref[...]
    acc = a[0]
    for m in range(1, NM):
        acc = acc | (a[m] << m)
    o_ref[...] = jnp.pad(acc, ((0, NOP - NO), (0, 0)))


def _bitpack(adj_t):
    return pl.pallas_call(
        _bp_body,
        grid=(B // _BPB,),
        in_specs=[pl.BlockSpec((NM, NO, _BPB), lambda r: (0, 0, r))],
        out_specs=pl.BlockSpec((NOP, _BPB), lambda r: (0, r)),
        out_shape=jax.ShapeDtypeStruct((NOP, B), jnp.int32),
    )(adj_t)


def _sc_body(bits_hbm, nop_hbm, mbu_hbm, trk_hbm, out_hbm,
             bbuf, nop_v, mbu_v, trk_v, out_v, s0):
    wid = lax.axis_index("s") * 2 + lax.axis_index("c")
    lane = lax.iota(jnp.int32, 16)

    coff = pl.multiple_of((wid // 4) * 128, 128)
    lbase = (wid % 4) * SPW
    cp_b = pltpu.make_async_copy(bits_hbm.at[:, pl.ds(coff, 128)], bbuf, s0)
    cp_n = pltpu.make_async_copy(nop_hbm.at[:, pl.ds(coff, 128)], nop_v, s0)
    cp_m = pltpu.make_async_copy(mbu_hbm.at[:, pl.ds(coff, 128)], mbu_v, s0)
    cp_t = pltpu.make_async_copy(trk_hbm.at[:, pl.ds(coff, 128)], trk_v, s0)
    cp_b.start()
    cp_n.start()
    cp_m.start()
    cp_t.start()
    cp_b.wait()
    cp_n.wait()
    cp_m.wait()
    cp_t.wait()

    def group(g, _):
        col = lbase + g * 16
        cvec = col + lane
        mb = [mbu_v[m, pl.ds(col, 16)] for m in range(NM)]

        def job_body(j, carry):
            bv, bf = carry
            ovec = nop_v[j, pl.ds(col, 16)]
            bw = plsc.load_gather(bbuf, [ovec, cvec])
            for m in range(NM):
                avail = (bw & (1 << m)) != 0
                upd = avail & (mb[m] < bv)
                bv = jnp.where(upd, mb[m], bv)
                bf = jnp.where(upd, j * NM + m, bf)
            return bv, bf

        bv0 = jnp.full((16,), jnp.float32(BIG))
        bf0 = jnp.zeros((16,), jnp.int32)
        bv, bf = lax.fori_loop(0, NJ, job_body, (bv0, bf0), unroll=2)

        btv = trk_v[0, pl.ds(col, 16)]
        bts = jnp.zeros((16,), jnp.int32)
        for t in range(1, NT):
            tvt = trk_v[t, pl.ds(col, 16)]
            tupd = tvt < btv
            btv = jnp.where(tupd, tvt, btv)
            bts = jnp.where(tupd, t, bts)

        act = 1 + (bf // NM) * (NM * NT) + (bf % NM) * NT + bts
        out_v[pl.ds(g * 16, 16)] = act
        return 0

    lax.fori_loop(0, SPW // 16, group, 0)
    pltpu.sync_copy(out_v, out_hbm.at[pl.ds(wid * SPW, SPW)])


_sc_fifo = functools.partial(
    pl.kernel,
    mesh=plsc.VectorSubcoreMesh(core_axis_name="c", subcore_axis_name="s"),
    out_type=jax.ShapeDtypeStruct((B,), jnp.int32),
    compiler_params=pltpu.CompilerParams(needs_layout_passes=False),
    scratch_types=[
        pltpu.VMEM((NOP, 128), jnp.int32),
        pltpu.VMEM((NJ, 128), jnp.int32),
        pltpu.VMEM((NM, 128), jnp.float32),
        pltpu.VMEM((NT, 128), jnp.float32),
        pltpu.VMEM((SPW,), jnp.int32),
        pltpu.SemaphoreType.DMA,
    ],
)(_sc_body)


_CS = 1024


def _oh_body(idx_ref, o_ref):
    r = pl.program_id(0)
    rows = lax.broadcasted_iota(jnp.int32, (_CS, B), 0) + r * _CS
    idx = idx_ref[pl.ds(0, 1), :]
    o_ref[...] = jnp.where(rows == idx, jnp.float32(1.0), jnp.float32(0.0))


def _onehot_t(act_idx):
    idx2 = jnp.broadcast_to(act_idx.reshape(1, B), (8, B))
    return pl.pallas_call(
        _oh_body,
        grid=(pl.cdiv(NA, _CS),),
        in_specs=[pl.BlockSpec((8, B), lambda r: (0, 0))],
        out_specs=pl.BlockSpec((_CS, B), lambda r: (r, 0)),
        out_shape=jax.ShapeDtypeStruct((NA, B), jnp.float32),
    )(idx2)


def kernel(job_done, machine_busy_until, truck_location, ops_ma_adj,
           next_op, truck_busy_until, action_mask):
    adj_t = jnp.transpose(ops_ma_adj, (1, 2, 0))
    bits_t = _bitpack(adj_t)
    act_idx = _sc_fifo(bits_t, next_op.T, machine_busy_until.T,
                       truck_busy_until.T)
    logits = _onehot_t(act_idx).T
    return (logits, action_mask)


# speedup vs baseline: 1.0010x; 1.0010x over previous
import functools

import jax
import jax.numpy as jnp
from jax import lax
from jax.experimental import pallas as pl
from jax.experimental.pallas import tpu as pltpu
from jax.experimental.pallas import tpu_sc as plsc

B = 1024
NJ = 100
NM = 20
NT = 10
NO = 500
NOP = 512
NA = 1 + NJ * NM * NT
NW = 32
SPW = B // NW
BIG = 1e9


_BPB = 256


def _bp_body(adj_ref, o_ref):
    a = adj_ref[...]
    acc = a[0]
    for m in range(1, NM):
        acc = acc | (a[m] << m)
    o_ref[...] = jnp.pad(acc, ((0, NOP - NO), (0, 0)))


def _bitpack(adj_t):
    return pl.pallas_call(
        _bp_body,
        grid=(B // _BPB,),
        in_specs=[pl.BlockSpec((NM, NO, _BPB), lambda r: (0, 0, r))],
        out_specs=pl.BlockSpec((NOP, _BPB), lambda r: (0, r)),
        out_shape=jax.ShapeDtypeStruct((NOP, B), jnp.int32),
    )(adj_t)


def _sc_body(bits_hbm, nop_hbm, mbu_hbm, trk_hbm, out_hbm,
             bbuf, nop_v, mbu_v, trk_v, out_v, s0):
    wid = lax.axis_index("s") * 2 + lax.axis_index("c")
    lane = lax.iota(jnp.int32, 16)

    coff = pl.multiple_of((wid // 4) * 128, 128)
    lbase = (wid % 4) * SPW
    cp_b = pltpu.make_async_copy(bits_hbm.at[:, pl.ds(coff, 128)], bbuf, s0)
    cp_n = pltpu.make_async_copy(nop_hbm.at[:, pl.ds(coff, 128)], nop_v, s0)
    cp_m = pltpu.make_async_copy(mbu_hbm.at[:, pl.ds(coff, 128)], mbu_v, s0)
    cp_t = pltpu.make_async_copy(trk_hbm.at[:, pl.ds(coff, 128)], trk_v, s0)
    cp_b.start()
    cp_n.start()
    cp_m.start()
    cp_t.start()
    cp_b.wait()
    cp_n.wait()
    cp_m.wait()
    cp_t.wait()

    def group(g, _):
        col = lbase + g * 16
        cvec = col + lane
        mb = [mbu_v[m, pl.ds(col, 16)] for m in range(NM)]

        def job_body(j, carry):
            bv, bf = carry
            ovec = nop_v[j, pl.ds(col, 16)]
            bw = plsc.load_gather(bbuf, [ovec, cvec])
            cand = [(jnp.where((bw & (1 << m)) != 0, mb[m], jnp.float32(BIG)),
                     jnp.full((16,), m)) for m in range(NM)]
            while len(cand) > 1:
                nxt = []
                for p in range(0, len(cand) - 1, 2):
                    (av, ai), (bvv, bi) = cand[p], cand[p + 1]
                    ta = av <= bvv
                    nxt.append((jnp.where(ta, av, bvv),
                                jnp.where(ta, ai, bi)))
                if len(cand) % 2:
                    nxt.append(cand[-1])
                cand = nxt
            jv, jm = cand[0]
            upd = jv < bv
            bv = jnp.where(upd, jv, bv)
            bf = jnp.where(upd, j * NM + jm, bf)
            return bv, bf

        bv0 = jnp.full((16,), jnp.float32(BIG))
        bf0 = jnp.zeros((16,), jnp.int32)
        bv, bf = lax.fori_loop(0, NJ, job_body, (bv0, bf0), unroll=2)

        btv = trk_v[0, pl.ds(col, 16)]
        bts = jnp.zeros((16,), jnp.int32)
        for t in range(1, NT):
            tvt = trk_v[t, pl.ds(col, 16)]
            tupd = tvt < btv
            btv = jnp.where(tupd, tvt, btv)
            bts = jnp.where(tupd, t, bts)

        act = 1 + (bf // NM) * (NM * NT) + (bf % NM) * NT + bts
        out_v[pl.ds(g * 16, 16)] = act
        return 0

    lax.fori_loop(0, SPW // 16, group, 0)
    pltpu.sync_copy(out_v, out_hbm.at[pl.ds(wid * SPW, SPW)])


_sc_fifo = functools.partial(
    pl.kernel,
    mesh=plsc.VectorSubcoreMesh(core_axis_name="c", subcore_axis_name="s"),
    out_type=jax.ShapeDtypeStruct((B,), jnp.int32),
    compiler_params=pltpu.CompilerParams(needs_layout_passes=False),
    scratch_types=[
        pltpu.VMEM((NOP, 128), jnp.int32),
        pltpu.VMEM((NJ, 128), jnp.int32),
        pltpu.VMEM((NM, 128), jnp.float32),
        pltpu.VMEM((NT, 128), jnp.float32),
        pltpu.VMEM((SPW,), jnp.int32),
        pltpu.SemaphoreType.DMA,
    ],
)(_sc_body)


_CS = 1024


def _oh_body(idx_ref, o_ref):
    r = pl.program_id(0)
    rows = lax.broadcasted_iota(jnp.int32, (_CS, B), 0) + r * _CS
    idx = idx_ref[pl.ds(0, 1), :]
    o_ref[...] = jnp.where(rows == idx, jnp.float32(1.0), jnp.float32(0.0))


def _onehot_t(act_idx):
    idx2 = jnp.broadcast_to(act_idx.reshape(1, B), (8, B))
    return pl.pallas_call(
        _oh_body,
        grid=(pl.cdiv(NA, _CS),),
        in_specs=[pl.BlockSpec((8, B), lambda r: (0, 0))],
        out_specs=pl.BlockSpec((_CS, B), lambda r: (r, 0)),
        out_shape=jax.ShapeDtypeStruct((NA, B), jnp.float32),
    )(idx2)


def kernel(job_done, machine_busy_until, truck_location, ops_ma_adj,
           next_op, truck_busy_until, action_mask):
    adj_t = jnp.transpose(ops_ma_adj, (1, 2, 0))
    bits_t = _bitpack(adj_t)
    act_idx = _sc_fifo(bits_t, next_op.T, machine_busy_until.T,
                       truck_busy_until.T)
    logits = _onehot_t(act_idx).T
    return (logits, action_mask)


# speedup vs baseline: 1.0280x; 1.0270x over previous
import functools

import jax
import jax.numpy as jnp
from jax import lax
from jax.experimental import pallas as pl
from jax.experimental.pallas import tpu as pltpu
from jax.experimental.pallas import tpu_sc as plsc

B = 1024
NJ = 100
NM = 20
NT = 10
NO = 500
NOP = 512
NA = 1 + NJ * NM * NT
NW = 32
SPW = B // NW
BIG = 1e9


_BPB = 256


def _bp_body(adj_ref, o_ref):
    a = adj_ref[...]
    acc = a[0]
    for m in range(1, NM):
        acc = acc | (a[m] << m)
    o_ref[...] = jnp.pad(acc, ((0, NOP - NO), (0, 0)))


def _bitpack(adj_t):
    return pl.pallas_call(
        _bp_body,
        grid=(B // _BPB,),
        in_specs=[pl.BlockSpec((NM, NO, _BPB), lambda r: (0, 0, r))],
        out_specs=pl.BlockSpec((NOP, _BPB), lambda r: (0, r)),
        out_shape=jax.ShapeDtypeStruct((NOP, B), jnp.int32),
    )(adj_t)


def _sc_body(bits_hbm, nop_hbm, mbu_hbm, trk_hbm, out_hbm,
             bbuf, nop_v, mbu_v, trk_v, out_v, s0):
    wid = lax.axis_index("s") * 2 + lax.axis_index("c")
    lane = lax.iota(jnp.int32, 16)

    coff = pl.multiple_of((wid // 4) * 128, 128)
    lbase = (wid % 4) * SPW
    cp_b = pltpu.make_async_copy(bits_hbm.at[:, pl.ds(coff, 128)], bbuf, s0)
    cp_n = pltpu.make_async_copy(nop_hbm.at[:, pl.ds(coff, 128)], nop_v, s0)
    cp_m = pltpu.make_async_copy(mbu_hbm.at[:, pl.ds(coff, 128)], mbu_v, s0)
    cp_t = pltpu.make_async_copy(trk_hbm.at[:, pl.ds(coff, 128)], trk_v, s0)
    cp_b.start()
    cp_n.start()
    cp_m.start()
    cp_t.start()
    cp_b.wait()
    cp_n.wait()
    cp_m.wait()
    cp_t.wait()

    def group(g, _):
        col = lbase + g * 16
        cvec = col + lane
        mb = [mbu_v[m, pl.ds(col, 16)] for m in range(NM)]

        def job_body(j, carry):
            bv, bf = carry
            ovec = nop_v[j, pl.ds(col, 16)]
            bw = plsc.load_gather(bbuf, [ovec, cvec])
            cand = [(jnp.where((bw & (1 << m)) != 0, mb[m], jnp.float32(BIG)),
                     jnp.full((16,), m)) for m in range(NM)]
            while len(cand) > 1:
                nxt = []
                for p in range(0, len(cand) - 1, 2):
                    (av, ai), (bvv, bi) = cand[p], cand[p + 1]
                    ta = av <= bvv
                    nxt.append((jnp.where(ta, av, bvv),
                                jnp.where(ta, ai, bi)))
                if len(cand) % 2:
                    nxt.append(cand[-1])
                cand = nxt
            jv, jm = cand[0]
            upd = jv < bv
            bv = jnp.where(upd, jv, bv)
            bf = jnp.where(upd, j * NM + jm, bf)
            return bv, bf

        bv0 = jnp.full((16,), jnp.float32(BIG))
        bf0 = jnp.zeros((16,), jnp.int32)
        bv, bf = lax.fori_loop(0, NJ, job_body, (bv0, bf0))

        btv = trk_v[0, pl.ds(col, 16)]
        bts = jnp.zeros((16,), jnp.int32)
        for t in range(1, NT):
            tvt = trk_v[t, pl.ds(col, 16)]
            tupd = tvt < btv
            btv = jnp.where(tupd, tvt, btv)
            bts = jnp.where(tupd, t, bts)

        act = 1 + (bf // NM) * (NM * NT) + (bf % NM) * NT + bts
        out_v[pl.ds(g * 16, 16)] = act
        return 0

    lax.fori_loop(0, SPW // 16, group, 0)
    pltpu.sync_copy(out_v, out_hbm.at[pl.ds(wid * SPW, SPW)])


_sc_fifo = functools.partial(
    pl.kernel,
    mesh=plsc.VectorSubcoreMesh(core_axis_name="c", subcore_axis_name="s"),
    out_type=jax.ShapeDtypeStruct((B,), jnp.int32),
    compiler_params=pltpu.CompilerParams(needs_layout_passes=False),
    scratch_types=[
        pltpu.VMEM((NOP, 128), jnp.int32),
        pltpu.VMEM((NJ, 128), jnp.int32),
        pltpu.VMEM((NM, 128), jnp.float32),
        pltpu.VMEM((NT, 128), jnp.float32),
        pltpu.VMEM((SPW,), jnp.int32),
        pltpu.SemaphoreType.DMA,
    ],
)(_sc_body)


_CS = 1024


def _oh_body(idx_ref, o_ref):
    r = pl.program_id(0)
    rows = lax.broadcasted_iota(jnp.int32, (_CS, B), 0) + r * _CS
    idx = idx_ref[pl.ds(0, 1), :]
    o_ref[...] = jnp.where(rows == idx, jnp.float32(1.0), jnp.float32(0.0))


def _onehot_t(act_idx):
    idx2 = jnp.broadcast_to(act_idx.reshape(1, B), (8, B))
    return pl.pallas_call(
        _oh_body,
        grid=(pl.cdiv(NA, _CS),),
        in_specs=[pl.BlockSpec((8, B), lambda r: (0, 0))],
        out_specs=pl.BlockSpec((_CS, B), lambda r: (r, 0)),
        out_shape=jax.ShapeDtypeStruct((NA, B), jnp.float32),
    )(idx2)


def kernel(job_done, machine_busy_until, truck_location, ops_ma_adj,
           next_op, truck_busy_until, action_mask):
    adj_t = jnp.transpose(ops_ma_adj, (1, 2, 0))
    bits_t = _bitpack(adj_t)
    act_idx = _sc_fifo(bits_t, next_op.T, machine_busy_until.T,
                       truck_busy_until.T)
    logits = _onehot_t(act_idx).T
    return (logits, action_mask)
